# baseline (speedup 1.0000x reference)
import functools

import jax
import jax.numpy as jnp
from jax.experimental import pallas as pl
from jax.experimental.pallas import tpu as pltpu

NP_TOT = 500000
NV = 65536
MAX_OUT = 102124
PAD_OUT = 102400
PT_BLK = 2000
N_PT_BLK = NP_TOT // PT_BLK
VOX_BLK = 2048


def _enc_mm1_body(params, feats, bidx, w1, b1, pf1_out, ids_out):
    x = feats[:, 0:1]
    y = feats[:, 1:2]
    z = feats[:, 2:3]
    rho = jnp.sqrt(x * x + y * y)
    phi = jnp.arctan2(y, x)
    c0 = params[0]
    inv_r = params[1]
    c2 = params[2]
    inv_p = params[3]
    c4 = params[4]
    inv_z = params[5]
    rho_n = (rho - c0) * inv_r
    phi_n = (phi - c2) * inv_p
    z_n = (z - c4) * inv_z
    r_idx = jnp.clip(jnp.floor(rho_n * 32.0), 0.0, 31.0).astype(jnp.int32)
    p_idx = jnp.clip(jnp.floor(phi_n * 32.0), 0.0, 31.0).astype(jnp.int32)
    z_idx = jnp.clip(jnp.floor(z_n * 16.0), 0.0, 15.0).astype(jnp.int32)
    b = bidx[:]
    ids_out[:] = ((b * 32 + r_idx) * 32 + p_idx) * 16 + z_idx
    f12 = jnp.concatenate([feats[:], rho_n, phi_n, z_n], axis=1)
    acc = jnp.dot(f12, w1[:], preferred_element_type=jnp.float32)
    pf1_out[:] = jnp.maximum(acc + b1[:], 0.0)


def _enc_mm1(params, feats, bidx2, W1, b1):
    return pl.pallas_call(
        _enc_mm1_body,
        grid=(N_PT_BLK,),
        in_specs=[
            pl.BlockSpec(memory_space=pltpu.SMEM),
            pl.BlockSpec((PT_BLK, 9), lambda i: (i, 0)),
            pl.BlockSpec((PT_BLK, 1), lambda i: (i, 0)),
            pl.BlockSpec((12, 64), lambda i: (0, 0)),
            pl.BlockSpec((1, 64), lambda i: (0, 0)),
        ],
        out_specs=[
            pl.BlockSpec((PT_BLK, 64), lambda i: (i, 0)),
            pl.BlockSpec((PT_BLK, 1), lambda i: (i, 0)),
        ],
        out_shape=[
            jax.ShapeDtypeStruct((NP_TOT, 64), jnp.float32),
            jax.ShapeDtypeStruct((NP_TOT, 1), jnp.int32),
        ],
    )(params, feats, bidx2, W1, b1)


def _mm_body(a, w, out):
    out[:] = jnp.dot(a[:], w[:], preferred_element_type=jnp.float32)


def _voxg(vox1, W2b):
    return pl.pallas_call(
        _mm_body,
        grid=(NV // VOX_BLK,),
        in_specs=[
            pl.BlockSpec((VOX_BLK, 64), lambda i: (i, 0)),
            pl.BlockSpec((64, 64), lambda i: (0, 0)),
        ],
        out_specs=pl.BlockSpec((VOX_BLK, 64), lambda i: (i, 0)),
        out_shape=jax.ShapeDtypeStruct((NV, 64), jnp.float32),
    )(vox1, W2b)


def _pf2_body(pf1, g, w2a, b2, out):
    acc = jnp.dot(pf1[:], w2a[:], preferred_element_type=jnp.float32)
    out[:] = jnp.maximum(acc + g[:] + b2[:], 0.0)


def _pf2(pf1, g, W2a, b2):
    return pl.pallas_call(
        _pf2_body,
        grid=(N_PT_BLK,),
        in_specs=[
            pl.BlockSpec((PT_BLK, 64), lambda i: (i, 0)),
            pl.BlockSpec((PT_BLK, 64), lambda i: (i, 0)),
            pl.BlockSpec((64, 64), lambda i: (0, 0)),
            pl.BlockSpec((1, 64), lambda i: (0, 0)),
        ],
        out_specs=pl.BlockSpec((PT_BLK, 64), lambda i: (i, 0)),
        out_shape=jax.ShapeDtypeStruct((NP_TOT, 64), jnp.float32),
    )(pf1, g, W2a, b2)


def _final_body(numact, voxc, occ_ids, w3, b3, feats_out, coors_out):
    i = pl.program_id(0)
    acc = jnp.dot(voxc[:], w3[:], preferred_element_type=jnp.float32)
    act = jnp.maximum(acc + b3[:], 0.0)
    pad = jnp.maximum(b3[:], 0.0)
    row = jax.lax.broadcasted_iota(jnp.int32, (VOX_BLK, 1), 0) + i * VOX_BLK
    mask = row < numact[0]
    feats_out[:] = jnp.where(mask, act, pad)
    uid = occ_ids[:]
    zc = jax.lax.rem(uid, 16)
    t = jax.lax.div(uid, 16)
    pc = jax.lax.rem(t, 32)
    t = jax.lax.div(t, 32)
    rc = jax.lax.rem(t, 32)
    bc = jax.lax.div(t, 32)
    coors_out[:] = jnp.concatenate([bc, rc, pc, zc], axis=1)


def _final(numact, voxc, occ_ids2, W3, b3):
    return pl.pallas_call(
        _final_body,
        grid=(PAD_OUT // VOX_BLK,),
        in_specs=[
            pl.BlockSpec(memory_space=pltpu.SMEM),
            pl.BlockSpec((VOX_BLK, 64), lambda i: (i, 0)),
            pl.BlockSpec((VOX_BLK, 1), lambda i: (i, 0)),
            pl.BlockSpec((64, 128), lambda i: (0, 0)),
            pl.BlockSpec((1, 128), lambda i: (0, 0)),
        ],
        out_specs=[
            pl.BlockSpec((VOX_BLK, 128), lambda i: (i, 0)),
            pl.BlockSpec((VOX_BLK, 4), lambda i: (i, 0)),
        ],
        out_shape=[
            jax.ShapeDtypeStruct((PAD_OUT, 128), jnp.float32),
            jax.ShapeDtypeStruct((PAD_OUT, 4), jnp.int32),
        ],
    )(numact, voxc, occ_ids2, W3, b3)


def kernel(batch_point_feats, batch_indices, cylinder_config, in_spatial_shape,
           W1, b1, W2, b2, W3, b3):
    cfg = cylinder_config
    params = jnp.stack([
        cfg[0], 1.0 / (cfg[1] - cfg[0]),
        cfg[2], 1.0 / (cfg[3] - cfg[2]),
        cfg[4], 1.0 / (cfg[5] - cfg[4]),
    ])
    bidx2 = batch_indices.reshape(NP_TOT, 1)
    pf1, ids2 = _enc_mm1(params, batch_point_feats, bidx2,
                         W1, b1.reshape(1, 64))
    ids = ids2.reshape(NP_TOT)

    occ = jnp.zeros((NV,), jnp.int32).at[ids].set(1)
    vox1 = jnp.zeros((NV, 64), jnp.float32).at[ids].max(pf1)

    voxg = _voxg(vox1, W2[64:])
    g = voxg[ids]
    pf2 = _pf2(pf1, g, W2[:64], b2.reshape(1, 64))

    vox2 = jnp.zeros((NV, 64), jnp.float32).at[ids].max(pf2)

    rank = jnp.cumsum(occ) - occ
    numact = jnp.sum(occ).astype(jnp.int32)
    occ_ids = jnp.zeros((PAD_OUT,), jnp.int32).at[
        jnp.where(occ == 1, rank, PAD_OUT)
    ].set(jnp.arange(NV, dtype=jnp.int32), mode="drop")
    voxc = vox2[occ_ids]

    feats_out, coors_out = _final(numact.reshape(1), voxc,
                                  occ_ids.reshape(PAD_OUT, 1), W3,
                                  b3.reshape(1, 128))
    return (feats_out[:MAX_OUT], coors_out[:MAX_OUT], numact)


# speedup vs baseline: 1.3815x; 1.3815x over previous
import functools

import jax
import jax.numpy as jnp
from jax import lax
from jax.experimental import pallas as pl
from jax.experimental.pallas import tpu as pltpu
from jax.experimental.pallas import tpu_sc as plsc

NP_TOT = 500000
NP_PAD = 524288
NV = 65536
NT = 67584
MAX_OUT = 102124
PAD_OUT = 131072
PT_BLK = 2000
N_PT_BLK = NP_TOT // PT_BLK
PT_BLK2 = 2048
VOX_BLK = 2048
NW = 32


def _enc_mm1_body(params, feats, bidx, w1, b1, pf1_out, ids_out):
    x = feats[:, 0:1]
    y = feats[:, 1:2]
    z = feats[:, 2:3]
    rho = jnp.sqrt(x * x + y * y)
    phi = jnp.arctan2(y, x)
    c0 = params[0]
    inv_r = params[1]
    c2 = params[2]
    inv_p = params[3]
    c4 = params[4]
    inv_z = params[5]
    rho_n = (rho - c0) * inv_r
    phi_n = (phi - c2) * inv_p
    z_n = (z - c4) * inv_z
    r_idx = jnp.clip(jnp.floor(rho_n * 32.0), 0.0, 31.0).astype(jnp.int32)
    p_idx = jnp.clip(jnp.floor(phi_n * 32.0), 0.0, 31.0).astype(jnp.int32)
    z_idx = jnp.clip(jnp.floor(z_n * 16.0), 0.0, 15.0).astype(jnp.int32)
    b = bidx[:]
    ids_out[:] = ((b * 32 + r_idx) * 32 + p_idx) * 16 + z_idx
    f12 = jnp.concatenate([feats[:], rho_n, phi_n, z_n], axis=1)
    acc = jnp.dot(f12, w1[:], preferred_element_type=jnp.float32)
    pf1_out[:] = jnp.maximum(acc + b1[:], 0.0)


def _enc_mm1(params, feats, bidx2, W1, b1):
    return pl.pallas_call(
        _enc_mm1_body,
        grid=(N_PT_BLK,),
        in_specs=[
            pl.BlockSpec(memory_space=pltpu.SMEM),
            pl.BlockSpec((PT_BLK, 9), lambda i: (i, 0)),
            pl.BlockSpec((PT_BLK, 1), lambda i: (i, 0)),
            pl.BlockSpec((12, 64), lambda i: (0, 0)),
            pl.BlockSpec((1, 64), lambda i: (0, 0)),
        ],
        out_specs=[
            pl.BlockSpec((PT_BLK, 64), lambda i: (i, 0)),
            pl.BlockSpec((PT_BLK, 1), lambda i: (i, 0)),
        ],
        out_shape=[
            jax.ShapeDtypeStruct((NP_PAD, 64), jnp.float32),
            jax.ShapeDtypeStruct((NP_TOT, 1), jnp.int32),
        ],
    )(params, feats, bidx2, W1, b1)


def _mm_body(a, w, b, out):
    acc = jnp.dot(a[:], w[:], preferred_element_type=jnp.float32)
    out[:] = acc + b[:]


def _mm_relu_body(a, w, b, out):
    acc = jnp.dot(a[:], w[:], preferred_element_type=jnp.float32)
    out[:] = jnp.maximum(acc + b[:], 0.0)


def _table_mm(vox, W, b, n_out, relu):
    return pl.pallas_call(
        _mm_relu_body if relu else _mm_body,
        grid=(NT // VOX_BLK,),
        in_specs=[
            pl.BlockSpec((VOX_BLK, 64), lambda i: (i, 0)),
            pl.BlockSpec((64, n_out), lambda i: (0, 0)),
            pl.BlockSpec((1, n_out), lambda i: (0, 0)),
        ],
        out_specs=pl.BlockSpec((VOX_BLK, n_out), lambda i: (i, 0)),
        out_shape=jax.ShapeDtypeStruct((NT, n_out), jnp.float32),
    )(vox, W, b)


def _pf2_body(pf1, g, w2a, out):
    acc = jnp.dot(pf1[:], w2a[:], preferred_element_type=jnp.float32)
    out[:] = jnp.maximum(acc + g[:, 0:64], 0.0)


def _pf2(pf1, g, W2a):
    return pl.pallas_call(
        _pf2_body,
        grid=(NP_PAD // PT_BLK2,),
        in_specs=[
            pl.BlockSpec((PT_BLK2, 64), lambda i: (i, 0)),
            pl.BlockSpec((PT_BLK2, 128), lambda i: (i, 0)),
            pl.BlockSpec((64, 64), lambda i: (0, 0)),
        ],
        out_specs=pl.BlockSpec((PT_BLK2, 64), lambda i: (i, 0)),
        out_shape=jax.ShapeDtypeStruct((NP_PAD, 64), jnp.float32),
    )(pf1, g, W2a)


def _make_sc_gather(rows_per_w, n_win):
    assert rows_per_w == n_win * 512
    idx_rows = rows_per_w // 128
    mesh = plsc.VectorSubcoreMesh(core_axis_name="c", subcore_axis_name="s")

    @functools.partial(
        pl.kernel, mesh=mesh,
        out_type=jax.ShapeDtypeStruct((rows_per_w * NW, 128), jnp.float32),
        scratch_types=[
            pltpu.VMEM((idx_rows, 128), jnp.int32),
            pltpu.VMEM((512, 128), jnp.float32),
            pltpu.SemaphoreType.DMA,
        ],
    )
    def k(table_hbm, ids_hbm, out_hbm, idx_v, rows_v, sem):
        wid = lax.axis_index("s") * 2 + lax.axis_index("c")
        pltpu.sync_copy(ids_hbm.at[wid], idx_v)

        def body(kw, carry):
            base = wid * rows_per_w + kw * 512
            cps = []
            for j in range(4):
                cps.append(pltpu.async_copy(
                    table_hbm.at[idx_v.at[kw * 4 + j]],
                    rows_v.at[pl.ds(j * 128, 128)], sem))
            for c in cps:
                c.wait()
            pltpu.sync_copy(rows_v, out_hbm.at[pl.ds(base, 512)])
            return carry

        lax.fori_loop(0, n_win, body, 0)

    return k


_sc_gather_pts = _make_sc_gather(16384, 32)
_sc_gather_vox = _make_sc_gather(4096, 8)


def _mask_body(numact, voxc, occ_ids, b3, feats_out, coors_out):
    i = pl.program_id(0)
    pad = jnp.maximum(b3[:], 0.0)
    row = jax.lax.broadcasted_iota(jnp.int32, (VOX_BLK, 1), 0) + i * VOX_BLK
    mask = row < numact[0]
    feats_out[:] = jnp.where(mask, voxc[:], pad)
    uid = occ_ids[:]
    zc = jax.lax.rem(uid, 16)
    t = jax.lax.div(uid, 16)
    pc = jax.lax.rem(t, 32)
    t = jax.lax.div(t, 32)
    rc = jax.lax.rem(t, 32)
    bc = jax.lax.div(t, 32)
    coors_out[:] = jnp.concatenate([bc, rc, pc, zc], axis=1)


def _mask_final(numact, voxc, occ_ids2, b3):
    return pl.pallas_call(
        _mask_body,
        grid=(PAD_OUT // VOX_BLK,),
        in_specs=[
            pl.BlockSpec(memory_space=pltpu.SMEM),
            pl.BlockSpec((VOX_BLK, 128), lambda i: (i, 0)),
            pl.BlockSpec((VOX_BLK, 1), lambda i: (i, 0)),
            pl.BlockSpec((1, 128), lambda i: (0, 0)),
        ],
        out_specs=[
            pl.BlockSpec((VOX_BLK, 128), lambda i: (i, 0)),
            pl.BlockSpec((VOX_BLK, 4), lambda i: (i, 0)),
        ],
        out_shape=[
            jax.ShapeDtypeStruct((PAD_OUT, 128), jnp.float32),
            jax.ShapeDtypeStruct((PAD_OUT, 4), jnp.int32),
        ],
    )(numact, voxc, occ_ids2, b3)


def kernel(batch_point_feats, batch_indices, cylinder_config, in_spatial_shape,
           W1, b1, W2, b2, W3, b3):
    cfg = cylinder_config
    params = jnp.stack([
        cfg[0], 1.0 / (cfg[1] - cfg[0]),
        cfg[2], 1.0 / (cfg[3] - cfg[2]),
        cfg[4], 1.0 / (cfg[5] - cfg[4]),
    ])
    bidx2 = batch_indices.reshape(NP_TOT, 1)
    pf1, ids2 = _enc_mm1(params, batch_point_feats, bidx2,
                         W1, b1.reshape(1, 64))
    ids = ids2.reshape(NP_TOT)
    pad_ids = NV + (jnp.arange(NP_PAD - NP_TOT, dtype=jnp.int32) % 2048)
    ids_g = jnp.concatenate([ids, pad_ids])

    occ = jnp.zeros((NV,), jnp.int32).at[ids].set(1, mode="drop")
    vox1 = jnp.zeros((NT, 64), jnp.float32).at[ids_g].max(pf1)

    W2b_pad = jnp.concatenate(
        [W2[64:], jnp.zeros((64, 64), jnp.float32)], axis=1)
    b2_pad = jnp.concatenate([b2, jnp.zeros((64,), jnp.float32)])
    voxg = _table_mm(vox1, W2b_pad, b2_pad.reshape(1, 128), 128, False)
    g = _sc_gather_pts(voxg, ids_g.reshape(NW, 128, 128))
    pf2 = _pf2(pf1, g, W2[:64])

    vox2 = jnp.zeros((NT, 64), jnp.float32).at[ids_g].max(pf2)
    dense_out = _table_mm(vox2, W3, b3.reshape(1, 128), 128, True)

    rank = jnp.cumsum(occ) - occ
    numact = jnp.sum(occ).astype(jnp.int32)
    occ_ids = jnp.zeros((PAD_OUT,), jnp.int32).at[
        jnp.where(occ == 1, rank, PAD_OUT)
    ].set(jnp.arange(NV, dtype=jnp.int32), mode="drop")
    arange_o = jnp.arange(PAD_OUT, dtype=jnp.int32)
    occ_ids_g = jnp.where(arange_o < numact, occ_ids, NV + (arange_o % 2048))
    voxc = _sc_gather_vox(dense_out, occ_ids_g.reshape(NW, 32, 128))

    feats_out, coors_out = _mask_final(numact.reshape(1), voxc,
                                       occ_ids.reshape(PAD_OUT, 1),
                                       b3.reshape(1, 128))
    return (feats_out[:MAX_OUT], coors_out[:MAX_OUT], numact)


# speedup vs baseline: 2.2685x; 1.6421x over previous
import functools

import jax
import jax.numpy as jnp
from jax import lax
from jax.experimental import pallas as pl
from jax.experimental.pallas import tpu as pltpu
from jax.experimental.pallas import tpu_sc as plsc

NP_TOT = 500000
NP_PAD = 524288
NV = 65536
NT = 67584
MAX_OUT = 102124
PAD_OUT = 131072
PT_BLK = 2000
N_PT_BLK = NP_TOT // PT_BLK
PT_BLK2 = 2048
VOX_BLK = 2048
NW = 32


def _enc_mm1_body(params, feats, bidx, w1, b1, w2a, pc_out, ids_out):
    x = feats[:, 0:1]
    y = feats[:, 1:2]
    z = feats[:, 2:3]
    rho = jnp.sqrt(x * x + y * y)
    phi = jnp.arctan2(y, x)
    c0 = params[0]
    inv_r = params[1]
    c2 = params[2]
    inv_p = params[3]
    c4 = params[4]
    inv_z = params[5]
    rho_n = (rho - c0) * inv_r
    phi_n = (phi - c2) * inv_p
    z_n = (z - c4) * inv_z
    r_idx = jnp.clip(jnp.floor(rho_n * 32.0), 0.0, 31.0).astype(jnp.int32)
    p_idx = jnp.clip(jnp.floor(phi_n * 32.0), 0.0, 31.0).astype(jnp.int32)
    z_idx = jnp.clip(jnp.floor(z_n * 16.0), 0.0, 15.0).astype(jnp.int32)
    b = bidx[:]
    ids_out[:] = ((b * 32 + r_idx) * 32 + p_idx) * 16 + z_idx
    f12 = jnp.concatenate([feats[:], rho_n, phi_n, z_n], axis=1)
    acc = jnp.dot(f12, w1[:], preferred_element_type=jnp.float32)
    pf1 = jnp.maximum(acc + b1[:], 0.0)
    h = jnp.dot(pf1, w2a[:], preferred_element_type=jnp.float32)
    pc_out[:] = jnp.concatenate([pf1, h], axis=1)


def _enc_mm1(params, feats, bidx2, W1, b1, W2a):
    return pl.pallas_call(
        _enc_mm1_body,
        grid=(N_PT_BLK,),
        in_specs=[
            pl.BlockSpec(memory_space=pltpu.SMEM),
            pl.BlockSpec((PT_BLK, 9), lambda i: (i, 0)),
            pl.BlockSpec((PT_BLK, 1), lambda i: (i, 0)),
            pl.BlockSpec((12, 64), lambda i: (0, 0)),
            pl.BlockSpec((1, 64), lambda i: (0, 0)),
            pl.BlockSpec((64, 64), lambda i: (0, 0)),
        ],
        out_specs=[
            pl.BlockSpec((PT_BLK, 128), lambda i: (i, 0)),
            pl.BlockSpec((PT_BLK, 1), lambda i: (i, 0)),
        ],
        out_shape=[
            jax.ShapeDtypeStruct((NP_PAD, 128), jnp.float32),
            jax.ShapeDtypeStruct((NP_TOT, 1), jnp.int32),
        ],
    )(params, feats, bidx2, W1, b1, W2a)


def _table_body(vox12, w2b, b2, w3, b3, dense_out, occ_out):
    occ_out[:] = (vox12[:, 64:65] > -1e30).astype(jnp.int32)
    vox1 = jnp.maximum(vox12[:, 0:64], 0.0)
    hmax = vox12[:, 64:128]
    voxg = jnp.dot(vox1, w2b[:], preferred_element_type=jnp.float32) + b2[:]
    vox2 = jnp.maximum(hmax + voxg, 0.0)
    acc = jnp.dot(vox2, w3[:], preferred_element_type=jnp.float32)
    dense_out[:] = jnp.maximum(acc + b3[:], 0.0)


def _table_pipeline(vox12, W2b, b2, W3, b3):
    return pl.pallas_call(
        _table_body,
        grid=(NT // VOX_BLK,),
        in_specs=[
            pl.BlockSpec((VOX_BLK, 128), lambda i: (i, 0)),
            pl.BlockSpec((64, 64), lambda i: (0, 0)),
            pl.BlockSpec((1, 64), lambda i: (0, 0)),
            pl.BlockSpec((64, 128), lambda i: (0, 0)),
            pl.BlockSpec((1, 128), lambda i: (0, 0)),
        ],
        out_specs=[
            pl.BlockSpec((VOX_BLK, 128), lambda i: (i, 0)),
            pl.BlockSpec((VOX_BLK, 1), lambda i: (i, 0)),
        ],
        out_shape=[
            jax.ShapeDtypeStruct((NT, 128), jnp.float32),
            jax.ShapeDtypeStruct((NT, 1), jnp.int32),
        ],
    )(vox12, W2b, b2, W3, b3)


def _make_sc_gather(rows_per_w, n_win):
    assert rows_per_w == n_win * 512
    idx_rows = rows_per_w // 128
    mesh = plsc.VectorSubcoreMesh(core_axis_name="c", subcore_axis_name="s")

    @functools.partial(
        pl.kernel, mesh=mesh,
        out_type=jax.ShapeDtypeStruct((rows_per_w * NW, 128), jnp.float32),
        scratch_types=[
            pltpu.VMEM((idx_rows, 128), jnp.int32),
            pltpu.VMEM((512, 128), jnp.float32),
            pltpu.SemaphoreType.DMA,
        ],
    )
    def k(table_hbm, ids_hbm, out_hbm, idx_v, rows_v, sem):
        wid = lax.axis_index("s") * 2 + lax.axis_index("c")
        pltpu.sync_copy(ids_hbm.at[wid], idx_v)

        def body(kw, carry):
            base = wid * rows_per_w + kw * 512
            cps = []
            for j in range(4):
                cps.append(pltpu.async_copy(
                    table_hbm.at[idx_v.at[kw * 4 + j]],
                    rows_v.at[pl.ds(j * 128, 128)], sem))
            for c in cps:
                c.wait()
            pltpu.sync_copy(rows_v, out_hbm.at[pl.ds(base, 512)])
            return carry

        lax.fori_loop(0, n_win, body, 0)

    return k


_sc_gather_vox = _make_sc_gather(4096, 8)


def _mask_body(numact, voxc, occ_ids, b3, feats_out, coors_out):
    i = pl.program_id(0)
    pad = jnp.maximum(b3[:], 0.0)
    row = jax.lax.broadcasted_iota(jnp.int32, (VOX_BLK, 1), 0) + i * VOX_BLK
    mask = row < numact[0]
    feats_out[:] = jnp.where(mask, voxc[:], pad)
    uid = occ_ids[:]
    zc = jax.lax.rem(uid, 16)
    t = jax.lax.div(uid, 16)
    pc = jax.lax.rem(t, 32)
    t = jax.lax.div(t, 32)
    rc = jax.lax.rem(t, 32)
    bc = jax.lax.div(t, 32)
    coors_out[:] = jnp.concatenate([bc, rc, pc, zc], axis=1)


def _mask_final(numact, voxc, occ_ids2, b3):
    return pl.pallas_call(
        _mask_body,
        grid=(PAD_OUT // VOX_BLK,),
        in_specs=[
            pl.BlockSpec(memory_space=pltpu.SMEM),
            pl.BlockSpec((VOX_BLK, 128), lambda i: (i, 0)),
            pl.BlockSpec((VOX_BLK, 1), lambda i: (i, 0)),
            pl.BlockSpec((1, 128), lambda i: (0, 0)),
        ],
        out_specs=[
            pl.BlockSpec((VOX_BLK, 128), lambda i: (i, 0)),
            pl.BlockSpec((VOX_BLK, 4), lambda i: (i, 0)),
        ],
        out_shape=[
            jax.ShapeDtypeStruct((PAD_OUT, 128), jnp.float32),
            jax.ShapeDtypeStruct((PAD_OUT, 4), jnp.int32),
        ],
    )(numact, voxc, occ_ids2, b3)


def kernel(batch_point_feats, batch_indices, cylinder_config, in_spatial_shape,
           W1, b1, W2, b2, W3, b3):
    cfg = cylinder_config
    params = jnp.stack([
        cfg[0], 1.0 / (cfg[1] - cfg[0]),
        cfg[2], 1.0 / (cfg[3] - cfg[2]),
        cfg[4], 1.0 / (cfg[5] - cfg[4]),
    ])
    bidx2 = batch_indices.reshape(NP_TOT, 1)
    pc, ids2 = _enc_mm1(params, batch_point_feats, bidx2,
                        W1, b1.reshape(1, 64), W2[:64])
    ids = ids2.reshape(NP_TOT)
    pad_ids = NV + (jnp.arange(NP_PAD - NP_TOT, dtype=jnp.int32) % 2048)
    ids_g = jnp.concatenate([ids, pad_ids])

    vox12 = jnp.full((NT, 128), -jnp.inf, jnp.float32).at[ids_g].max(pc)
    dense_out, occ2 = _table_pipeline(vox12, W2[64:], b2.reshape(1, 64),
                                      W3, b3.reshape(1, 128))
    occ = occ2[:NV, 0]

    rank = jnp.cumsum(occ) - occ
    numact = jnp.sum(occ).astype(jnp.int32)
    occ_ids = jnp.zeros((PAD_OUT,), jnp.int32).at[
        jnp.where(occ == 1, rank, PAD_OUT)
    ].set(jnp.arange(NV, dtype=jnp.int32), mode="drop")
    arange_o = jnp.arange(PAD_OUT, dtype=jnp.int32)
    occ_ids_g = jnp.where(arange_o < numact, occ_ids, NV + (arange_o % 2048))
    voxc = _sc_gather_vox(dense_out, occ_ids_g.reshape(NW, 32, 128))

    feats_out, coors_out = _mask_final(numact.reshape(1), voxc,
                                       occ_ids.reshape(PAD_OUT, 1),
                                       b3.reshape(1, 128))
    return (feats_out[:MAX_OUT], coors_out[:MAX_OUT], numact)
